# baseline (speedup 1.0000x reference)
import functools

import jax
import jax.numpy as jnp
from jax.experimental import pallas as pl


def _proj_kernel(x_ref, w_ref, b_ref, o_ref):
    o_ref[...] = jax.nn.relu(
        jnp.dot(x_ref[...], w_ref[...], preferred_element_type=jnp.float32)
        + b_ref[...]
    )


def _proj(x, W, b):
    n, fi = x.shape
    fo = W.shape[1]
    blk = 5000
    grid = n // blk
    return pl.pallas_call(
        _proj_kernel,
        grid=(grid,),
        in_specs=[
            pl.BlockSpec((blk, fi), lambda i: (i, 0)),
            pl.BlockSpec((fi, fo), lambda i: (0, 0)),
            pl.BlockSpec((fo,), lambda i: (0,)),
        ],
        out_specs=pl.BlockSpec((blk, fo), lambda i: (i, 0)),
        out_shape=jax.ShapeDtypeStruct((n, fo), jnp.float32),
    )(x, W, b)


def _gatv2(x, src, dst, Wl, Wr, att, bias, n):
    xl = x @ Wl
    xr = x @ Wr
    z = jax.nn.leaky_relu(xl[dst] + xr[src], negative_slope=0.2)
    logits = z @ att
    m = jax.ops.segment_max(logits, dst, num_segments=n)
    m = jnp.where(jnp.isfinite(m), m, 0.0)
    ex = jnp.exp(logits - m[dst])
    denom = jax.ops.segment_sum(ex, dst, num_segments=n)
    alpha = ex / (denom[dst] + 1e-16)
    out = jax.ops.segment_sum(xr[src] * alpha[:, None], dst, num_segments=n)
    return out + bias


def kernel(x1, x2, attack_armies, deploy_armies, params, edges, attack_src, attack_dst, deploy_target):
    p = params
    n = x1.shape[0]
    src, dst = edges[0], edges[1]
    x2t = jnp.tile(x2, (n, 1))
    x_ = _proj(x1, p['W0'], p['b0'])
    xa = jax.nn.relu(_gatv2(jnp.concatenate([x_, x1], axis=1), src, dst, p['Wl1'], p['Wr1'], p['att1'], p['bias1'], n))
    xb = jax.nn.relu(_gatv2(jnp.concatenate([xa, x_, x1], axis=1), src, dst, p['Wl2'], p['Wr2'], p['att2'], p['bias2'], n))
    xc = jax.nn.relu(_gatv2(jnp.concatenate([xb, xa, x_, x1], axis=1), src, dst, p['Wl3'], p['Wr3'], p['att3'], p['bias3'], n))

    def per_move(asrc, adst, aarm, dtgt, darm):
        extra = jnp.stack([aarm, 0.6 * aarm - 0.7 * (x1[adst, 3] + x1[adst, 4])], axis=1)
        at = jnp.concatenate([xc[asrc], xc[adst], x1[asrc, 3:], x1[adst, 1:], extra], axis=1)
        at = at @ p['Wa'] + p['ba']
        dt = jnp.concatenate([xc[dtgt], x1[dtgt, 3:], darm[:, None]], axis=1)
        dt = dt @ p['Wd'] + p['bd']
        ot = jax.nn.relu(jnp.concatenate([at, dt], axis=0))
        attn = jax.nn.softmax(ot @ p['Woa'] + p['boa'], axis=0)
        return jnp.sum(attn * (ot @ p['Wov'] + p['bov']))

    pv = jax.vmap(per_move)(attack_src, attack_dst, attack_armies, deploy_target, deploy_armies)
    V = jnp.concatenate([xc, x1, x2t], axis=1)
    V = jax.nn.relu(V @ p['Wv1'] + p['bv1'])
    attn = jax.nn.softmax(V @ p['Wva'] + p['bva'], axis=0)
    V = jax.nn.relu(jnp.sum(attn * (V @ p['Wvv'] + p['bvv']), axis=0))
    V = jnp.tanh(V @ p['Wvl'] + p['bvl']).reshape(())
    return (V, jax.nn.log_softmax(pv, axis=0))


# speedup vs baseline: 16.3130x; 16.3130x over previous
import functools

import jax
import jax.numpy as jnp
from jax import lax
from jax.experimental import pallas as pl
from jax.experimental.pallas import tpu as pltpu
from jax.experimental.pallas import tpu_sc as plsc

_BLK = 5000
_EPB = 80
_NW = 32
_NPAD = 51200
_EBLK = 2000


def _tcA_body(x1_ref, w0_ref, b0_ref, wlr_ref, xp_ref, xl_ref, xr_ref):
    xb = x1_ref[...]
    blk = xb.shape[0]
    x_ = jax.nn.relu(jnp.dot(xb, w0_ref[...], preferred_element_type=jnp.float32)
                     + b0_ref[...])
    h = jnp.concatenate([x_, xb], axis=1)
    y = jnp.dot(h, wlr_ref[...], preferred_element_type=jnp.float32)
    z6 = jnp.zeros((blk, 6), jnp.float32)
    xp_ref[...] = jnp.concatenate([x_, z6], axis=1)
    xl_ref[...] = jnp.concatenate([y[:, :10], z6], axis=1)
    xr_ref[...] = jnp.concatenate(
        [y[:, 10:], jnp.ones((blk, 1), jnp.float32), jnp.zeros((blk, 5), jnp.float32)],
        axis=1)


def _tc_first(x1, W0, b0, Wlr):
    n = x1.shape[0]
    grid = n // _BLK
    outs = [jax.ShapeDtypeStruct((n, 16), jnp.float32)] * 3
    return pl.pallas_call(
        _tcA_body,
        grid=(grid,),
        in_specs=[
            pl.BlockSpec((_BLK, 15), lambda i: (i, 0)),
            pl.BlockSpec((15, 10), lambda i: (0, 0)),
            pl.BlockSpec((10,), lambda i: (0,)),
            pl.BlockSpec((25, 20), lambda i: (0, 0)),
        ],
        out_specs=[pl.BlockSpec((_BLK, 16), lambda i: (i, 0))] * 3,
        out_shape=outs,
    )(x1, W0, b0, Wlr)


def _tc_mid_body(n_prev, a0_ref, a1_ref, bias_ref, x1_ref, *rest):
    prev_refs = rest[:n_prev]
    wlr_ref = rest[n_prev]
    xout_ref, xl_ref, xr_ref = rest[n_prev + 1:]
    a = a0_ref[...] + a1_ref[...]
    blk = a.shape[0]
    xa = jax.nn.relu(a[:, :10] / (a[:, 10:11] + 1e-16) + bias_ref[...])
    h = jnp.concatenate(
        [xa] + [r[...][:, :10] for r in prev_refs] + [x1_ref[...]], axis=1)
    y = jnp.dot(h, wlr_ref[...], preferred_element_type=jnp.float32)
    z6 = jnp.zeros((blk, 6), jnp.float32)
    xout_ref[...] = jnp.concatenate([xa, z6], axis=1)
    xl_ref[...] = jnp.concatenate([y[:, :10], z6], axis=1)
    xr_ref[...] = jnp.concatenate(
        [y[:, 10:], jnp.ones((blk, 1), jnp.float32), jnp.zeros((blk, 5), jnp.float32)],
        axis=1)


def _tc_mid(a0, a1, bias, x1, prevs, Wlr):
    n = x1.shape[0]
    grid = n // _BLK
    fi = Wlr.shape[0]
    outs = [jax.ShapeDtypeStruct((n, 16), jnp.float32)] * 3
    in_specs = [
        pl.BlockSpec((_BLK, 16), lambda i: (i, 0)),
        pl.BlockSpec((_BLK, 16), lambda i: (i, 0)),
        pl.BlockSpec((10,), lambda i: (0,)),
        pl.BlockSpec((_BLK, 15), lambda i: (i, 0)),
    ] + [pl.BlockSpec((_BLK, 16), lambda i: (i, 0)) for _ in prevs] + [
        pl.BlockSpec((fi, 20), lambda i: (0, 0)),
    ]
    return pl.pallas_call(
        functools.partial(_tc_mid_body, len(prevs)),
        grid=(grid,),
        in_specs=in_specs,
        out_specs=[pl.BlockSpec((_BLK, 16), lambda i: (i, 0))] * 3,
        out_shape=outs,
    )(a0, a1, bias, x1, *prevs, Wlr)


def _tc_edge_body(l_ref, r_ref, att_ref, v_ref):
    l = l_ref[...]
    r = r_ref[...]
    s = l + r
    t = (0.6 * s + 0.4 * jnp.abs(s)) * att_ref[...]
    grp = lax.broadcasted_iota(jnp.int32, (128, 8), 0) // 16
    col = lax.broadcasted_iota(jnp.int32, (128, 8), 1)
    S = (grp == col).astype(jnp.float32)
    grp2 = lax.broadcasted_iota(jnp.int32, (8, 128), 1) // 16
    row2 = lax.broadcasted_iota(jnp.int32, (8, 128), 0)
    St = (grp2 == row2).astype(jnp.float32)
    logits = jnp.dot(t, S, preferred_element_type=jnp.float32)
    ex = jnp.exp(logits)
    v_ref[...] = r * jnp.dot(ex, St, preferred_element_type=jnp.float32)


def _tc_edge(lp, rp, att128):
    m = lp.shape[0]
    grid = m // _EBLK
    return pl.pallas_call(
        _tc_edge_body,
        grid=(grid,),
        in_specs=[
            pl.BlockSpec((_EBLK, 128), lambda i: (i, 0)),
            pl.BlockSpec((_EBLK, 128), lambda i: (i, 0)),
            pl.BlockSpec((1, 128), lambda i: (0, 0)),
        ],
        out_specs=pl.BlockSpec((_EBLK, 128), lambda i: (i, 0)),
        out_shape=jax.ShapeDtypeStruct((m, 128), jnp.float32),
    )(lp, rp, att128)


def _tcD_body(a0_ref, a1_ref, bias_ref, x1_ref, x2_ref,
              wv1_ref, bv1_ref, wva_ref, bva_ref, wvv_ref, bvv_ref,
              xc_ref, part_ref):
    a = a0_ref[...] + a1_ref[...]
    blk = a.shape[0]
    xc = jax.nn.relu(a[:, :10] / (a[:, 10:11] + 1e-16) + bias_ref[...])
    xc_ref[...] = jnp.concatenate([xc, jnp.zeros((blk, 6), jnp.float32)], axis=1)
    h = jnp.concatenate(
        [xc, x1_ref[...], jnp.broadcast_to(x2_ref[...], (blk, 4))], axis=1)
    V1 = jax.nn.relu(jnp.dot(h, wv1_ref[...], preferred_element_type=jnp.float32)
                     + bv1_ref[...])
    s = jnp.dot(V1, wva_ref[...], preferred_element_type=jnp.float32) + bva_ref[...]
    se = jnp.exp(s)
    sv = jnp.dot(V1, wvv_ref[...], preferred_element_type=jnp.float32) + bvv_ref[...]
    psum = jnp.sum(se * sv, axis=0, keepdims=True)
    tot = jnp.sum(se, axis=0, keepdims=True)
    part = jnp.concatenate([psum, tot, jnp.zeros((1, 5), jnp.float32)], axis=1)
    part_ref[...] = part.reshape(1, 1, 16)


def _tc_value(a0, a1, bias, x1, x2, p):
    n = x1.shape[0]
    grid = n // _BLK
    outs = [jax.ShapeDtypeStruct((n, 16), jnp.float32),
            jax.ShapeDtypeStruct((grid, 1, 16), jnp.float32)]
    return pl.pallas_call(
        _tcD_body,
        grid=(grid,),
        in_specs=[
            pl.BlockSpec((_BLK, 16), lambda i: (i, 0)),
            pl.BlockSpec((_BLK, 16), lambda i: (i, 0)),
            pl.BlockSpec((10,), lambda i: (0,)),
            pl.BlockSpec((_BLK, 15), lambda i: (i, 0)),
            pl.BlockSpec((1, 4), lambda i: (0, 0)),
            pl.BlockSpec((29, 20), lambda i: (0, 0)),
            pl.BlockSpec((20,), lambda i: (0,)),
            pl.BlockSpec((20, 1), lambda i: (0, 0)),
            pl.BlockSpec((1,), lambda i: (0,)),
            pl.BlockSpec((20, 10), lambda i: (0, 0)),
            pl.BlockSpec((10,), lambda i: (0,)),
        ],
        out_specs=[pl.BlockSpec((_BLK, 16), lambda i: (i, 0)),
                   pl.BlockSpec((1, 1, 16), lambda i: (i, 0, 0))],
        out_shape=outs,
    )(a0, a1, bias, x1, x2, p['Wv1'], p['bv1'], p['Wva'], p['bva'],
      p['Wvv'], p['bvv'])


def _tcE_body(parts_ref, gxc_ref, gx1_ref, aarm_ref, darm_ref,
              wvl_ref, bvl_ref, wa_ref, ba_ref, wd_ref, bd_ref,
              woa_ref, boa_ref, wov_ref, bov_ref,
              outv_ref, outpv_ref):
    parts = parts_ref[...][:, 0, :]
    sv = jnp.sum(parts[:, :10], axis=0, keepdims=True)
    tot = jnp.sum(parts[:, 10])
    V10 = jax.nn.relu(sv / tot)
    Vs = jnp.tanh(jnp.dot(V10, wvl_ref[...],
                          preferred_element_type=jnp.float32) + bvl_ref[...])
    outv_ref[...] = Vs

    gxc = gxc_ref[...]
    gx1 = gx1_ref[...]
    asrc_c = gxc[0:512].reshape(16, 32, 16)[..., :10]
    adst_c = gxc[512:1024].reshape(16, 32, 16)[..., :10]
    dtgt_c = gxc[1024:1280].reshape(16, 16, 16)[..., :10]
    asrc_1 = gx1[0:512].reshape(16, 32, 16)
    adst_1 = gx1[512:1024].reshape(16, 32, 16)
    dtgt_1 = gx1[1024:1280].reshape(16, 16, 16)
    aarm = aarm_ref[...]
    darm = darm_ref[...]
    extra2 = 0.6 * aarm - 0.7 * (adst_1[..., 3] + adst_1[..., 4])
    at = jnp.concatenate(
        [asrc_c, adst_c, asrc_1[..., 3:15], adst_1[..., 1:15],
         aarm[..., None], extra2[..., None]], axis=2)
    at = (jnp.dot(at.reshape(512, 48), wa_ref[...],
                  preferred_element_type=jnp.float32) + ba_ref[...]).reshape(16, 32, 20)
    dt = jnp.concatenate(
        [dtgt_c, dtgt_1[..., 3:15], darm[..., None]], axis=2)
    dt = (jnp.dot(dt.reshape(256, 23), wd_ref[...],
                  preferred_element_type=jnp.float32) + bd_ref[...]).reshape(16, 16, 20)
    ot = jax.nn.relu(jnp.concatenate([at, dt], axis=1))
    ot2 = ot.reshape(768, 20)
    oa = (jnp.dot(ot2, woa_ref[...],
                  preferred_element_type=jnp.float32) + boa_ref[...]).reshape(16, 48)
    ov = (jnp.dot(ot2, wov_ref[...],
                  preferred_element_type=jnp.float32) + bov_ref[...]).reshape(16, 48)
    attn = jax.nn.softmax(oa, axis=1)
    pv = jnp.sum(attn * ov, axis=1, keepdims=True).reshape(1, 16)
    outpv_ref[...] = jax.nn.log_softmax(pv, axis=1)


def _tc_heads(parts, gxc, gx1, aarm, darm, p):
    outs = [jax.ShapeDtypeStruct((1, 1), jnp.float32),
            jax.ShapeDtypeStruct((1, 16), jnp.float32)]
    return pl.pallas_call(
        _tcE_body,
        out_shape=outs,
    )(parts, gxc, gx1, aarm, darm,
      p['Wvl'], p['bvl'], p['Wa'], p['ba'], p['Wd'], p['bd'],
      p['Woa'], p['boa'], p['Wov'], p['bov'])


def _edge_gather(src, dst, xl, xr):
    e = src.shape[0]
    epw = e // _NW
    nblk = epw // _EPB
    mesh = plsc.VectorSubcoreMesh(core_axis_name="c", subcore_axis_name="s")

    @functools.partial(
        pl.kernel,
        mesh=mesh,
        compiler_params=pltpu.CompilerParams(use_tc_tiling_on_sc=False),
        out_type=[jax.ShapeDtypeStruct((e * 16,), jnp.float32),
                  jax.ShapeDtypeStruct((e * 16,), jnp.float32)],
        scratch_types=[
            pltpu.VMEM((_EPB,), jnp.int32),
            pltpu.VMEM((_EPB,), jnp.int32),
            pltpu.VMEM((_EPB, 16), jnp.float32),
            pltpu.VMEM((_EPB, 16), jnp.float32),
            pltpu.VMEM((_EPB * 16,), jnp.float32),
            pltpu.VMEM((_EPB * 16,), jnp.float32),
            pltpu.SemaphoreType.DMA,
            pltpu.SemaphoreType.DMA,
        ],
    )
    def gk(src_h, dst_h, xl_h, xr_h, outl_h, outr_h,
           srcb, dstb, rows_l, rows_r, lflat, rflat, sem1, sem2):
        cid = lax.axis_index("c")
        sid = lax.axis_index("s")
        wid = sid * 2 + cid
        base0 = wid * epw

        def blk(i, carry):
            base = pl.multiple_of(base0 + i * _EPB, 16)
            pltpu.sync_copy(src_h.at[pl.ds(base, _EPB)], srcb)
            pltpu.sync_copy(dst_h.at[pl.ds(base, _EPB)], dstb)
            g1 = pltpu.async_copy(xl_h.at[dstb], rows_l, sem1)
            g2 = pltpu.async_copy(xr_h.at[srcb], rows_r, sem2)
            g1.wait()
            g2.wait()
            for j in range(_EPB):
                lflat[pl.ds(j * 16, 16)] = rows_l[j]
                rflat[pl.ds(j * 16, 16)] = rows_r[j]
            fbase = pl.multiple_of((base0 + i * _EPB) * 16, 128)
            pltpu.sync_copy(lflat, outl_h.at[pl.ds(fbase, _EPB * 16)])
            pltpu.sync_copy(rflat, outr_h.at[pl.ds(fbase, _EPB * 16)])
            return carry
        lax.fori_loop(0, nblk, blk, 0)

    return gk(src, dst, xl, xr)


def _edge_scatter(dst, vflat, n):
    e = dst.shape[0]
    epw = e // _NW
    nblk = epw // _EPB
    rows_sub = _NPAD // 16
    chunk = 400
    nchunk = rows_sub // chunk
    mesh = plsc.VectorSubcoreMesh(core_axis_name="c", subcore_axis_name="s")

    @functools.partial(
        pl.kernel,
        mesh=mesh,
        compiler_params=pltpu.CompilerParams(use_tc_tiling_on_sc=False),
        out_type=jax.ShapeDtypeStruct((2 * _NPAD, 16), jnp.float32),
        scratch_types=[
            pltpu.VMEM((_EPB,), jnp.int32),
            pltpu.VMEM((_EPB * 16,), jnp.float32),
            pltpu.VMEM((_EPB, 16), jnp.float32),
            pltpu.VMEM((chunk, 16), jnp.float32),
            pltpu.VMEM_SHARED((_NPAD, 16), jnp.float32),
            pltpu.SemaphoreType.DMA,
        ],
    )
    def sk(dst_h, v_h, out_h, dstb, vbuf, sbuf, stage, acc_sh, sem1):
        cid = lax.axis_index("c")
        sid = lax.axis_index("s")
        wid = sid * 2 + cid

        zv = jnp.zeros((16,), jnp.float32)

        def zrow(r, carry):
            stage[r] = zv
            return carry
        lax.fori_loop(0, chunk, zrow, 0)

        def zchunk(c, carry):
            r0 = pl.multiple_of(sid * rows_sub + c * chunk, 8)
            pltpu.sync_copy(stage, acc_sh.at[pl.ds(r0, chunk)])
            return carry
        lax.fori_loop(0, nchunk, zchunk, 0)
        plsc.subcore_barrier()

        base0 = wid * epw

        def blk(i, carry):
            base = pl.multiple_of(base0 + i * _EPB, 16)
            pltpu.sync_copy(dst_h.at[pl.ds(base, _EPB)], dstb)
            fbase = pl.multiple_of((base0 + i * _EPB) * 16, 128)
            pltpu.sync_copy(v_h.at[pl.ds(fbase, _EPB * 16)], vbuf)
            for j in range(_EPB):
                sbuf[j] = vbuf[pl.ds(j * 16, 16)]
            pltpu.sync_copy(sbuf, acc_sh.at[dstb], add=True)
            return carry
        lax.fori_loop(0, nblk, blk, 0)
        plsc.subcore_barrier()

        def dchunk(c, carry):
            r0 = pl.multiple_of(sid * rows_sub + c * chunk, 8)
            pltpu.sync_copy(acc_sh.at[pl.ds(r0, chunk)], stage)
            pltpu.sync_copy(stage, out_h.at[pl.ds(cid * _NPAD + r0, chunk)])
            return carry
        lax.fori_loop(0, nchunk, dchunk, 0)

    return sk(dst, vflat)


def _move_gather(idx, xcp, x1p):
    b = idx.shape[0]
    per_w = b // _NW
    mesh = plsc.VectorSubcoreMesh(core_axis_name="c", subcore_axis_name="s")

    @functools.partial(
        pl.kernel,
        mesh=mesh,
        compiler_params=pltpu.CompilerParams(use_tc_tiling_on_sc=False),
        out_type=[jax.ShapeDtypeStruct((b, 16), jnp.float32),
                  jax.ShapeDtypeStruct((b, 16), jnp.float32)],
        scratch_types=[
            pltpu.VMEM((per_w,), jnp.int32),
            pltpu.VMEM((per_w, 16), jnp.float32),
            pltpu.VMEM((per_w, 16), jnp.float32),
            pltpu.SemaphoreType.DMA,
            pltpu.SemaphoreType.DMA,
        ],
    )
    def mg(idx_h, xc_h, x1_h, oxc_h, ox1_h, idxv, r1, r2, sem1, sem2):
        cid = lax.axis_index("c")
        sid = lax.axis_index("s")
        wid = sid * 2 + cid
        base = pl.multiple_of(wid * per_w, 8)
        pltpu.sync_copy(idx_h.at[pl.ds(base, per_w)], idxv)
        a = pltpu.async_copy(xc_h.at[idxv], r1, sem1)
        bb = pltpu.async_copy(x1_h.at[idxv], r2, sem2)
        a.wait()
        bb.wait()
        pltpu.sync_copy(r1, oxc_h.at[pl.ds(base, per_w)])
        pltpu.sync_copy(r2, ox1_h.at[pl.ds(base, per_w)])

    return mg(idx, xcp, x1p)


def _edge_pass(src, dst, xl, xr, att128, n):
    e = src.shape[0]
    lf, rf = _edge_gather(src, dst, xl, xr)
    lp = lf.reshape(e // 8, 128)
    rp = rf.reshape(e // 8, 128)
    v = _tc_edge(lp, rp, att128)
    acc = _edge_scatter(dst, v.reshape(e * 16), n)
    return acc


def kernel(x1, x2, attack_armies, deploy_armies, params, edges, attack_src,
           attack_dst, deploy_target):
    p = params
    n = x1.shape[0]
    src = edges[0]
    dst = edges[1]

    Wlr1 = jnp.concatenate([p['Wl1'], p['Wr1']], axis=1)
    Wlr2 = jnp.concatenate([p['Wl2'], p['Wr2']], axis=1)
    Wlr3 = jnp.concatenate([p['Wl3'], p['Wr3']], axis=1)
    att1p = jnp.tile(jnp.pad(p['att1'], (0, 6)), 8).reshape(1, 128)
    att2p = jnp.tile(jnp.pad(p['att2'], (0, 6)), 8).reshape(1, 128)
    att3p = jnp.tile(jnp.pad(p['att3'], (0, 6)), 8).reshape(1, 128)
    x1p = jnp.pad(x1, ((0, 0), (0, 1)))

    xp, xl1, xr1 = _tc_first(x1, p['W0'], p['b0'], Wlr1)
    acc1 = _edge_pass(src, dst, xl1, xr1, att1p, n)
    xap, xl2, xr2 = _tc_mid(acc1[:n], acc1[_NPAD:_NPAD + n], p['bias1'],
                            x1, [xp], Wlr2)
    acc2 = _edge_pass(src, dst, xl2, xr2, att2p, n)
    xbp, xl3, xr3 = _tc_mid(acc2[:n], acc2[_NPAD:_NPAD + n], p['bias2'],
                            x1, [xap, xp], Wlr3)
    acc3 = _edge_pass(src, dst, xl3, xr3, att3p, n)

    xcp, parts = _tc_value(acc3[:n], acc3[_NPAD:_NPAD + n], p['bias3'],
                           x1, x2, p)

    midx = jnp.concatenate([attack_src.reshape(-1), attack_dst.reshape(-1),
                            deploy_target.reshape(-1)])
    gxc, gx1 = _move_gather(midx, xcp, x1p)

    outv, outpv = _tc_heads(parts, gxc, gx1, attack_armies, deploy_armies, p)
    return (outv.reshape(()), outpv.reshape(16))


# speedup vs baseline: 52.6369x; 3.2267x over previous
import functools

import jax
import jax.numpy as jnp
from jax import lax
from jax.experimental import pallas as pl
from jax.experimental.pallas import tpu as pltpu
from jax.experimental.pallas import tpu_sc as plsc

_BLK = 5000
_EPB = 80
_NW = 32
_NPAD = 51200
_EBLK = 2000


def _tcA_body(x1_ref, w0_ref, b0_ref, wlr_ref, xp_ref, xl_ref, xr_ref):
    xb = x1_ref[...]
    blk = xb.shape[0]
    x_ = jax.nn.relu(jnp.dot(xb, w0_ref[...], preferred_element_type=jnp.float32)
                     + b0_ref[...])
    h = jnp.concatenate([x_, xb], axis=1)
    y = jnp.dot(h, wlr_ref[...], preferred_element_type=jnp.float32)
    z6 = jnp.zeros((blk, 6), jnp.float32)
    xp_ref[...] = jnp.concatenate([x_, z6], axis=1)
    xl_ref[...] = jnp.concatenate([y[:, :10], z6], axis=1)
    xr_ref[...] = jnp.concatenate(
        [y[:, 10:], jnp.ones((blk, 1), jnp.float32), jnp.zeros((blk, 5), jnp.float32)],
        axis=1)


def _tc_first(x1, W0, b0, Wlr):
    n = x1.shape[0]
    grid = n // _BLK
    outs = [jax.ShapeDtypeStruct((n, 16), jnp.float32)] * 3
    return pl.pallas_call(
        _tcA_body,
        grid=(grid,),
        in_specs=[
            pl.BlockSpec((_BLK, 15), lambda i: (i, 0)),
            pl.BlockSpec((15, 10), lambda i: (0, 0)),
            pl.BlockSpec((10,), lambda i: (0,)),
            pl.BlockSpec((25, 20), lambda i: (0, 0)),
        ],
        out_specs=[pl.BlockSpec((_BLK, 16), lambda i: (i, 0))] * 3,
        out_shape=outs,
    )(x1, W0, b0, Wlr)


def _tc_mid_body(n_prev, a0_ref, a1_ref, bias_ref, x1_ref, *rest):
    prev_refs = rest[:n_prev]
    wlr_ref = rest[n_prev]
    xout_ref, xl_ref, xr_ref = rest[n_prev + 1:]
    a = a0_ref[...] + a1_ref[...]
    blk = a.shape[0]
    xa = jax.nn.relu(a[:, :10] / (a[:, 10:11] + 1e-16) + bias_ref[...])
    h = jnp.concatenate(
        [xa] + [r[...][:, :10] for r in prev_refs] + [x1_ref[...]], axis=1)
    y = jnp.dot(h, wlr_ref[...], preferred_element_type=jnp.float32)
    z6 = jnp.zeros((blk, 6), jnp.float32)
    xout_ref[...] = jnp.concatenate([xa, z6], axis=1)
    xl_ref[...] = jnp.concatenate([y[:, :10], z6], axis=1)
    xr_ref[...] = jnp.concatenate(
        [y[:, 10:], jnp.ones((blk, 1), jnp.float32), jnp.zeros((blk, 5), jnp.float32)],
        axis=1)


def _tc_mid(a0, a1, bias, x1, prevs, Wlr):
    n = x1.shape[0]
    grid = n // _BLK
    fi = Wlr.shape[0]
    outs = [jax.ShapeDtypeStruct((n, 16), jnp.float32)] * 3
    in_specs = [
        pl.BlockSpec((_BLK, 16), lambda i: (i, 0)),
        pl.BlockSpec((_BLK, 16), lambda i: (i, 0)),
        pl.BlockSpec((10,), lambda i: (0,)),
        pl.BlockSpec((_BLK, 15), lambda i: (i, 0)),
    ] + [pl.BlockSpec((_BLK, 16), lambda i: (i, 0)) for _ in prevs] + [
        pl.BlockSpec((fi, 20), lambda i: (0, 0)),
    ]
    return pl.pallas_call(
        functools.partial(_tc_mid_body, len(prevs)),
        grid=(grid,),
        in_specs=in_specs,
        out_specs=[pl.BlockSpec((_BLK, 16), lambda i: (i, 0))] * 3,
        out_shape=outs,
    )(a0, a1, bias, x1, *prevs, Wlr)


def _tc_edge_body(l_ref, r_ref, att_ref, v_ref):
    l = l_ref[...]
    r = r_ref[...]
    s = l + r
    t = (0.6 * s + 0.4 * jnp.abs(s)) * att_ref[...]
    grp = lax.broadcasted_iota(jnp.int32, (128, 8), 0) // 16
    col = lax.broadcasted_iota(jnp.int32, (128, 8), 1)
    S = (grp == col).astype(jnp.float32)
    grp2 = lax.broadcasted_iota(jnp.int32, (8, 128), 1) // 16
    row2 = lax.broadcasted_iota(jnp.int32, (8, 128), 0)
    St = (grp2 == row2).astype(jnp.float32)
    logits = jnp.dot(t, S, preferred_element_type=jnp.float32)
    ex = jnp.exp(logits)
    v_ref[...] = r * jnp.dot(ex, St, preferred_element_type=jnp.float32)


def _tc_edge(lp, rp, att128):
    m = lp.shape[0]
    grid = m // _EBLK
    return pl.pallas_call(
        _tc_edge_body,
        grid=(grid,),
        in_specs=[
            pl.BlockSpec((_EBLK, 128), lambda i: (i, 0)),
            pl.BlockSpec((_EBLK, 128), lambda i: (i, 0)),
            pl.BlockSpec((1, 128), lambda i: (0, 0)),
        ],
        out_specs=pl.BlockSpec((_EBLK, 128), lambda i: (i, 0)),
        out_shape=jax.ShapeDtypeStruct((m, 128), jnp.float32),
    )(lp, rp, att128)


def _tcD_body(a0_ref, a1_ref, bias_ref, x1_ref, x2_ref,
              wv1_ref, bv1_ref, wva_ref, bva_ref, wvv_ref, bvv_ref,
              xc_ref, part_ref):
    a = a0_ref[...] + a1_ref[...]
    blk = a.shape[0]
    xc = jax.nn.relu(a[:, :10] / (a[:, 10:11] + 1e-16) + bias_ref[...])
    xc_ref[...] = jnp.concatenate([xc, jnp.zeros((blk, 6), jnp.float32)], axis=1)
    h = jnp.concatenate(
        [xc, x1_ref[...], jnp.broadcast_to(x2_ref[...], (blk, 4))], axis=1)
    V1 = jax.nn.relu(jnp.dot(h, wv1_ref[...], preferred_element_type=jnp.float32)
                     + bv1_ref[...])
    s = jnp.dot(V1, wva_ref[...], preferred_element_type=jnp.float32) + bva_ref[...]
    se = jnp.exp(s)
    sv = jnp.dot(V1, wvv_ref[...], preferred_element_type=jnp.float32) + bvv_ref[...]
    psum = jnp.sum(se * sv, axis=0, keepdims=True)
    tot = jnp.sum(se, axis=0, keepdims=True)
    part = jnp.concatenate([psum, tot, jnp.zeros((1, 5), jnp.float32)], axis=1)
    part_ref[...] = part.reshape(1, 1, 16)


def _tc_value(a0, a1, bias, x1, x2, p):
    n = x1.shape[0]
    grid = n // _BLK
    outs = [jax.ShapeDtypeStruct((n, 16), jnp.float32),
            jax.ShapeDtypeStruct((grid, 1, 16), jnp.float32)]
    return pl.pallas_call(
        _tcD_body,
        grid=(grid,),
        in_specs=[
            pl.BlockSpec((_BLK, 16), lambda i: (i, 0)),
            pl.BlockSpec((_BLK, 16), lambda i: (i, 0)),
            pl.BlockSpec((10,), lambda i: (0,)),
            pl.BlockSpec((_BLK, 15), lambda i: (i, 0)),
            pl.BlockSpec((1, 4), lambda i: (0, 0)),
            pl.BlockSpec((29, 20), lambda i: (0, 0)),
            pl.BlockSpec((20,), lambda i: (0,)),
            pl.BlockSpec((20, 1), lambda i: (0, 0)),
            pl.BlockSpec((1,), lambda i: (0,)),
            pl.BlockSpec((20, 10), lambda i: (0, 0)),
            pl.BlockSpec((10,), lambda i: (0,)),
        ],
        out_specs=[pl.BlockSpec((_BLK, 16), lambda i: (i, 0)),
                   pl.BlockSpec((1, 1, 16), lambda i: (i, 0, 0))],
        out_shape=outs,
    )(a0, a1, bias, x1, x2, p['Wv1'], p['bv1'], p['Wva'], p['bva'],
      p['Wvv'], p['bvv'])


def _tcE_body(parts_ref, gxc_ref, gx1_ref, aarm_ref, darm_ref,
              wvl_ref, bvl_ref, wa_ref, ba_ref, wd_ref, bd_ref,
              woa_ref, boa_ref, wov_ref, bov_ref,
              outv_ref, outpv_ref):
    parts = parts_ref[...][:, 0, :]
    sv = jnp.sum(parts[:, :10], axis=0, keepdims=True)
    tot = jnp.sum(parts[:, 10])
    V10 = jax.nn.relu(sv / tot)
    Vs = jnp.tanh(jnp.dot(V10, wvl_ref[...],
                          preferred_element_type=jnp.float32) + bvl_ref[...])
    outv_ref[...] = Vs

    gxc = gxc_ref[...]
    gx1 = gx1_ref[...]
    asrc_c = gxc[0:512].reshape(16, 32, 16)[..., :10]
    adst_c = gxc[512:1024].reshape(16, 32, 16)[..., :10]
    dtgt_c = gxc[1024:1280].reshape(16, 16, 16)[..., :10]
    asrc_1 = gx1[0:512].reshape(16, 32, 16)
    adst_1 = gx1[512:1024].reshape(16, 32, 16)
    dtgt_1 = gx1[1024:1280].reshape(16, 16, 16)
    aarm = aarm_ref[...]
    darm = darm_ref[...]
    extra2 = 0.6 * aarm - 0.7 * (adst_1[..., 3] + adst_1[..., 4])
    at = jnp.concatenate(
        [asrc_c, adst_c, asrc_1[..., 3:15], adst_1[..., 1:15],
         aarm[..., None], extra2[..., None]], axis=2)
    at = (jnp.dot(at.reshape(512, 48), wa_ref[...],
                  preferred_element_type=jnp.float32) + ba_ref[...]).reshape(16, 32, 20)
    dt = jnp.concatenate(
        [dtgt_c, dtgt_1[..., 3:15], darm[..., None]], axis=2)
    dt = (jnp.dot(dt.reshape(256, 23), wd_ref[...],
                  preferred_element_type=jnp.float32) + bd_ref[...]).reshape(16, 16, 20)
    ot = jax.nn.relu(jnp.concatenate([at, dt], axis=1))
    ot2 = ot.reshape(768, 20)
    oa = (jnp.dot(ot2, woa_ref[...],
                  preferred_element_type=jnp.float32) + boa_ref[...]).reshape(16, 48)
    ov = (jnp.dot(ot2, wov_ref[...],
                  preferred_element_type=jnp.float32) + bov_ref[...]).reshape(16, 48)
    attn = jax.nn.softmax(oa, axis=1)
    pv = jnp.sum(attn * ov, axis=1, keepdims=True).reshape(1, 16)
    outpv_ref[...] = jax.nn.log_softmax(pv, axis=1)


def _tc_heads(parts, gxc, gx1, aarm, darm, p):
    outs = [jax.ShapeDtypeStruct((1, 1), jnp.float32),
            jax.ShapeDtypeStruct((1, 16), jnp.float32)]
    return pl.pallas_call(
        _tcE_body,
        out_shape=outs,
    )(parts, gxc, gx1, aarm, darm,
      p['Wvl'], p['bvl'], p['Wa'], p['ba'], p['Wd'], p['bd'],
      p['Woa'], p['boa'], p['Wov'], p['bov'])


_CBLK = 25
_NS = 5


def _edge_gather(src, dst, xl, xr):
    e = src.shape[0]
    epw = e // _NW
    nblk = epw // _EPB
    nchunks = nblk // _CBLK
    cbe = _CBLK * _EPB
    mesh = plsc.VectorSubcoreMesh(core_axis_name="c", subcore_axis_name="s")

    scratch = [
        pltpu.VMEM((cbe,), jnp.int32),
        pltpu.VMEM((cbe,), jnp.int32),
    ]
    scratch += [pltpu.VMEM((_EPB, 16), jnp.float32) for _ in range(2 * _NS)]
    scratch += [pltpu.VMEM((_EPB * 16,), jnp.float32) for _ in range(2 * _NS)]
    scratch += [pltpu.SemaphoreType.DMA for _ in range(2 * _NS)]

    @functools.partial(
        pl.kernel,
        mesh=mesh,
        compiler_params=pltpu.CompilerParams(use_tc_tiling_on_sc=False),
        out_type=[jax.ShapeDtypeStruct((e * 16,), jnp.float32),
                  jax.ShapeDtypeStruct((e * 16,), jnp.float32)],
        scratch_types=scratch,
    )
    def gk(src_h, dst_h, xl_h, xr_h, outl_h, outr_h, *scr):
        srcs_big, dsts_big = scr[0], scr[1]
        rows_l = list(scr[2:2 + _NS])
        rows_r = list(scr[2 + _NS:2 + 2 * _NS])
        fl = list(scr[2 + 2 * _NS:2 + 3 * _NS])
        fr = list(scr[2 + 3 * _NS:2 + 4 * _NS])
        sem_g = list(scr[2 + 4 * _NS:2 + 5 * _NS])
        sem_w = list(scr[2 + 5 * _NS:2 + 6 * _NS])
        cid = lax.axis_index("c")
        sid = lax.axis_index("s")
        wid = sid * 2 + cid
        base0 = wid * epw

        def issue_gather(b, k):
            o = pl.multiple_of(k * _EPB, 16)
            g1 = pltpu.async_copy(xl_h.at[dsts_big.at[pl.ds(o, _EPB)]],
                                  rows_l[b], sem_g[b])
            g2 = pltpu.async_copy(xr_h.at[srcs_big.at[pl.ds(o, _EPB)]],
                                  rows_r[b], sem_g[b])
            return g1, g2

        def wait_gather(b):
            pltpu.make_async_copy(xl_h.at[dsts_big.at[pl.ds(0, _EPB)]],
                                  rows_l[b], sem_g[b]).wait()
            pltpu.make_async_copy(xr_h.at[srcs_big.at[pl.ds(0, _EPB)]],
                                  rows_r[b], sem_g[b]).wait()

        def wait_writeout(b):
            pltpu.make_async_copy(fl[b], outl_h.at[pl.ds(0, _EPB * 16)],
                                  sem_w[b]).wait()
            pltpu.make_async_copy(fr[b], outr_h.at[pl.ds(0, _EPB * 16)],
                                  sem_w[b]).wait()

        def chunk(c, carry):
            cb = pl.multiple_of(base0 + c * cbe, 16)
            pltpu.sync_copy(src_h.at[pl.ds(cb, cbe)], srcs_big)
            pltpu.sync_copy(dst_h.at[pl.ds(cb, cbe)], dsts_big)
            for b in range(_NS):
                issue_gather(b, b)

            def group(g, carry2):
                for b in range(_NS):
                    k = g * _NS + b
                    gi = c * _CBLK + k
                    wait_gather(b)
                    @pl.when(gi >= _NS)
                    def _():
                        wait_writeout(b)
                    for j in range(_EPB):
                        fl[b][pl.ds(j * 16, 16)] = rows_l[b][j]
                        fr[b][pl.ds(j * 16, 16)] = rows_r[b][j]
                    fbase = pl.multiple_of((base0 + gi * _EPB) * 16, 128)
                    pltpu.async_copy(fl[b], outl_h.at[pl.ds(fbase, _EPB * 16)],
                                     sem_w[b])
                    pltpu.async_copy(fr[b], outr_h.at[pl.ds(fbase, _EPB * 16)],
                                     sem_w[b])
                    @pl.when(k + _NS < _CBLK)
                    def _():
                        issue_gather(b, k + _NS)
                return carry2
            lax.fori_loop(0, _CBLK // _NS, group, 0)
            return carry
        lax.fori_loop(0, nchunks, chunk, 0)
        for b in range(_NS):
            wait_writeout(b)

    return gk(src, dst, xl, xr)


def _edge_scatter(dst, vflat, n):
    e = dst.shape[0]
    epw = e // _NW
    nblk = epw // _EPB
    nchunks = nblk // _CBLK
    rows_sub = _NPAD // 16
    chunk_rows = 400
    nchunk = rows_sub // chunk_rows
    mesh = plsc.VectorSubcoreMesh(core_axis_name="c", subcore_axis_name="s")

    scratch = [pltpu.VMEM((_CBLK * _EPB,), jnp.int32)]
    scratch += [pltpu.VMEM((_EPB * 16,), jnp.float32) for _ in range(_NS)]
    scratch += [pltpu.VMEM((_EPB, 16), jnp.float32) for _ in range(_NS)]
    scratch += [pltpu.VMEM((_EPB,), jnp.int32) for _ in range(_NS)]
    scratch += [
        pltpu.VMEM((chunk_rows, 16), jnp.float32),
        pltpu.VMEM_SHARED((_NPAD, 16), jnp.float32),
    ]
    scratch += [pltpu.SemaphoreType.DMA for _ in range(2 * _NS)]

    @functools.partial(
        pl.kernel,
        mesh=mesh,
        compiler_params=pltpu.CompilerParams(use_tc_tiling_on_sc=False),
        out_type=jax.ShapeDtypeStruct((2 * _NPAD, 16), jnp.float32),
        scratch_types=scratch,
    )
    def sk(dst_h, v_h, out_h, *scr):
        dsts_big = scr[0]
        vbuf = list(scr[1:1 + _NS])
        sbuf = list(scr[1 + _NS:1 + 2 * _NS])
        dstc = list(scr[1 + 2 * _NS:1 + 3 * _NS])
        stage = scr[1 + 3 * _NS]
        acc_sh = scr[2 + 3 * _NS]
        sem_v = list(scr[3 + 3 * _NS:3 + 4 * _NS])
        sem_s = list(scr[3 + 4 * _NS:3 + 5 * _NS])
        cid = lax.axis_index("c")
        sid = lax.axis_index("s")
        wid = sid * 2 + cid

        zv = jnp.zeros((16,), jnp.float32)

        def zrow(r, carry):
            stage[r] = zv
            return carry
        lax.fori_loop(0, chunk_rows, zrow, 0)

        def zchunk(c, carry):
            r0 = pl.multiple_of(sid * rows_sub + c * chunk_rows, 8)
            pltpu.sync_copy(stage, acc_sh.at[pl.ds(r0, chunk_rows)])
            return carry
        lax.fori_loop(0, nchunk, zchunk, 0)
        plsc.subcore_barrier()

        base0 = wid * epw

        def issue_vload(b, gi):
            fbase = pl.multiple_of((base0 + gi * _EPB) * 16, 128)
            pltpu.async_copy(v_h.at[pl.ds(fbase, _EPB * 16)], vbuf[b], sem_v[b])

        def wait_vload(b):
            pltpu.make_async_copy(v_h.at[pl.ds(0, _EPB * 16)], vbuf[b],
                                  sem_v[b]).wait()

        def wait_scatter(b):
            pltpu.make_async_copy(sbuf[b], acc_sh.at[dstc[b]], sem_s[b]).wait()

        def chunk(c, carry):
            cb = pl.multiple_of(base0 + c * _CBLK * _EPB, 16)
            pltpu.sync_copy(dst_h.at[pl.ds(cb, _CBLK * _EPB)], dsts_big)
            for b in range(_NS):
                issue_vload(b, c * _CBLK + b)

            def group(g, carry2):
                for b in range(_NS):
                    k = g * _NS + b
                    gi = c * _CBLK + k
                    wait_vload(b)
                    @pl.when(gi >= _NS)
                    def _():
                        wait_scatter(b)
                    for j in range(_EPB):
                        sbuf[b][j] = vbuf[b][pl.ds(j * 16, 16)]
                    ko = pl.multiple_of(k * _EPB, 16)
                    for j5 in range(_EPB // 16):
                        dstc[b][pl.ds(j5 * 16, 16)] = (
                            dsts_big[pl.ds(ko + j5 * 16, 16)])
                    pltpu.async_copy(sbuf[b], acc_sh.at[dstc[b]], sem_s[b],
                                     add=True)
                    @pl.when(k + _NS < _CBLK)
                    def _():
                        issue_vload(b, gi + _NS)
                return carry2
            lax.fori_loop(0, _CBLK // _NS, group, 0)
            return carry
        lax.fori_loop(0, nchunks, chunk, 0)
        for b in range(_NS):
            wait_scatter(b)
        plsc.subcore_barrier()

        def dchunk(c, carry):
            r0 = pl.multiple_of(sid * rows_sub + c * chunk_rows, 8)
            pltpu.sync_copy(acc_sh.at[pl.ds(r0, chunk_rows)], stage)
            pltpu.sync_copy(stage, out_h.at[pl.ds(cid * _NPAD + r0, chunk_rows)])
            return carry
        lax.fori_loop(0, nchunk, dchunk, 0)

    return sk(dst, vflat)


def _move_gather(idx, xcp, x1p):
    b = idx.shape[0]
    per_w = b // _NW
    mesh = plsc.VectorSubcoreMesh(core_axis_name="c", subcore_axis_name="s")

    @functools.partial(
        pl.kernel,
        mesh=mesh,
        compiler_params=pltpu.CompilerParams(use_tc_tiling_on_sc=False),
        out_type=[jax.ShapeDtypeStruct((b, 16), jnp.float32),
                  jax.ShapeDtypeStruct((b, 16), jnp.float32)],
        scratch_types=[
            pltpu.VMEM((per_w,), jnp.int32),
            pltpu.VMEM((per_w, 16), jnp.float32),
            pltpu.VMEM((per_w, 16), jnp.float32),
            pltpu.SemaphoreType.DMA,
            pltpu.SemaphoreType.DMA,
        ],
    )
    def mg(idx_h, xc_h, x1_h, oxc_h, ox1_h, idxv, r1, r2, sem1, sem2):
        cid = lax.axis_index("c")
        sid = lax.axis_index("s")
        wid = sid * 2 + cid
        base = pl.multiple_of(wid * per_w, 8)
        pltpu.sync_copy(idx_h.at[pl.ds(base, per_w)], idxv)
        a = pltpu.async_copy(xc_h.at[idxv], r1, sem1)
        bb = pltpu.async_copy(x1_h.at[idxv], r2, sem2)
        a.wait()
        bb.wait()
        pltpu.sync_copy(r1, oxc_h.at[pl.ds(base, per_w)])
        pltpu.sync_copy(r2, ox1_h.at[pl.ds(base, per_w)])

    return mg(idx, xcp, x1p)


def _edge_pass(src, dst, xl, xr, att128, n):
    e = src.shape[0]
    lf, rf = _edge_gather(src, dst, xl, xr)
    lp = lf.reshape(e // 8, 128)
    rp = rf.reshape(e // 8, 128)
    v = _tc_edge(lp, rp, att128)
    acc = _edge_scatter(dst, v.reshape(e * 16), n)
    return acc


def kernel(x1, x2, attack_armies, deploy_armies, params, edges, attack_src,
           attack_dst, deploy_target):
    p = params
    n = x1.shape[0]
    src = edges[0]
    dst = edges[1]

    Wlr1 = jnp.concatenate([p['Wl1'], p['Wr1']], axis=1)
    Wlr2 = jnp.concatenate([p['Wl2'], p['Wr2']], axis=1)
    Wlr3 = jnp.concatenate([p['Wl3'], p['Wr3']], axis=1)
    att1p = jnp.tile(jnp.pad(p['att1'], (0, 6)), 8).reshape(1, 128)
    att2p = jnp.tile(jnp.pad(p['att2'], (0, 6)), 8).reshape(1, 128)
    att3p = jnp.tile(jnp.pad(p['att3'], (0, 6)), 8).reshape(1, 128)
    x1p = jnp.pad(x1, ((0, 0), (0, 1)))

    xp, xl1, xr1 = _tc_first(x1, p['W0'], p['b0'], Wlr1)
    acc1 = _edge_pass(src, dst, xl1, xr1, att1p, n)
    xap, xl2, xr2 = _tc_mid(acc1[:n], acc1[_NPAD:_NPAD + n], p['bias1'],
                            x1, [xp], Wlr2)
    acc2 = _edge_pass(src, dst, xl2, xr2, att2p, n)
    xbp, xl3, xr3 = _tc_mid(acc2[:n], acc2[_NPAD:_NPAD + n], p['bias2'],
                            x1, [xap, xp], Wlr3)
    acc3 = _edge_pass(src, dst, xl3, xr3, att3p, n)

    xcp, parts = _tc_value(acc3[:n], acc3[_NPAD:_NPAD + n], p['bias3'],
                           x1, x2, p)

    midx = jnp.concatenate([attack_src.reshape(-1), attack_dst.reshape(-1),
                            deploy_target.reshape(-1)])
    gxc, gx1 = _move_gather(midx, xcp, x1p)

    outv, outpv = _tc_heads(parts, gxc, gx1, attack_armies, deploy_armies, p)
    return (outv.reshape(()), outpv.reshape(16))


# speedup vs baseline: 60.9016x; 1.1570x over previous
import functools

import jax
import jax.numpy as jnp
from jax import lax
from jax.experimental import pallas as pl
from jax.experimental.pallas import tpu as pltpu
from jax.experimental.pallas import tpu_sc as plsc

_BLK = 5000
_EPB = 80
_NW = 32
_NPAD = 51200
_EBLK = 2000


def _tcA_body(x1_ref, w0_ref, b0_ref, wlr_ref, xp_ref, xl_ref, xr_ref):
    xb = x1_ref[...]
    blk = xb.shape[0]
    x_ = jax.nn.relu(jnp.dot(xb, w0_ref[...], preferred_element_type=jnp.float32)
                     + b0_ref[...])
    h = jnp.concatenate([x_, xb], axis=1)
    y = jnp.dot(h, wlr_ref[...], preferred_element_type=jnp.float32)
    z6 = jnp.zeros((blk, 6), jnp.float32)
    xp_ref[...] = jnp.concatenate([x_, z6], axis=1)
    xl_ref[...] = jnp.concatenate([y[:, :10], z6], axis=1)
    xr_ref[...] = jnp.concatenate(
        [y[:, 10:], jnp.ones((blk, 1), jnp.float32), jnp.zeros((blk, 5), jnp.float32)],
        axis=1)


def _tc_first(x1, W0, b0, Wlr):
    n = x1.shape[0]
    grid = n // _BLK
    outs = [jax.ShapeDtypeStruct((n, 16), jnp.float32)] * 3
    return pl.pallas_call(
        _tcA_body,
        grid=(grid,),
        in_specs=[
            pl.BlockSpec((_BLK, 15), lambda i: (i, 0)),
            pl.BlockSpec((15, 10), lambda i: (0, 0)),
            pl.BlockSpec((10,), lambda i: (0,)),
            pl.BlockSpec((25, 20), lambda i: (0, 0)),
        ],
        out_specs=[pl.BlockSpec((_BLK, 16), lambda i: (i, 0))] * 3,
        out_shape=outs,
    )(x1, W0, b0, Wlr)


def _tc_mid_body(n_prev, a0_ref, a1_ref, bias_ref, x1_ref, *rest):
    prev_refs = rest[:n_prev]
    wlr_ref = rest[n_prev]
    xout_ref, xl_ref, xr_ref = rest[n_prev + 1:]
    a = a0_ref[...] + a1_ref[...]
    blk = a.shape[0]
    xa = jax.nn.relu(a[:, :10] / (a[:, 10:11] + 1e-16) + bias_ref[...])
    h = jnp.concatenate(
        [xa] + [r[...][:, :10] for r in prev_refs] + [x1_ref[...]], axis=1)
    y = jnp.dot(h, wlr_ref[...], preferred_element_type=jnp.float32)
    z6 = jnp.zeros((blk, 6), jnp.float32)
    xout_ref[...] = jnp.concatenate([xa, z6], axis=1)
    xl_ref[...] = jnp.concatenate([y[:, :10], z6], axis=1)
    xr_ref[...] = jnp.concatenate(
        [y[:, 10:], jnp.ones((blk, 1), jnp.float32), jnp.zeros((blk, 5), jnp.float32)],
        axis=1)


def _tc_mid(a0, a1, bias, x1, prevs, Wlr):
    n = x1.shape[0]
    grid = n // _BLK
    fi = Wlr.shape[0]
    outs = [jax.ShapeDtypeStruct((n, 16), jnp.float32)] * 3
    in_specs = [
        pl.BlockSpec((_BLK, 16), lambda i: (i, 0)),
        pl.BlockSpec((_BLK, 16), lambda i: (i, 0)),
        pl.BlockSpec((10,), lambda i: (0,)),
        pl.BlockSpec((_BLK, 15), lambda i: (i, 0)),
    ] + [pl.BlockSpec((_BLK, 16), lambda i: (i, 0)) for _ in prevs] + [
        pl.BlockSpec((fi, 20), lambda i: (0, 0)),
    ]
    return pl.pallas_call(
        functools.partial(_tc_mid_body, len(prevs)),
        grid=(grid,),
        in_specs=in_specs,
        out_specs=[pl.BlockSpec((_BLK, 16), lambda i: (i, 0))] * 3,
        out_shape=outs,
    )(a0, a1, bias, x1, *prevs, Wlr)


def _tc_edge_body(l_ref, r_ref, att_ref, v_ref):
    l = l_ref[...]
    r = r_ref[...]
    s = l + r
    t = (0.6 * s + 0.4 * jnp.abs(s)) * att_ref[...]
    grp = lax.broadcasted_iota(jnp.int32, (128, 8), 0) // 16
    col = lax.broadcasted_iota(jnp.int32, (128, 8), 1)
    S = (grp == col).astype(jnp.float32)
    grp2 = lax.broadcasted_iota(jnp.int32, (8, 128), 1) // 16
    row2 = lax.broadcasted_iota(jnp.int32, (8, 128), 0)
    St = (grp2 == row2).astype(jnp.float32)
    logits = jnp.dot(t, S, preferred_element_type=jnp.float32)
    ex = jnp.exp(logits)
    v_ref[...] = r * jnp.dot(ex, St, preferred_element_type=jnp.float32)


def _tc_edge(lp, rp, att128):
    m = lp.shape[0]
    grid = m // _EBLK
    return pl.pallas_call(
        _tc_edge_body,
        grid=(grid,),
        in_specs=[
            pl.BlockSpec((_EBLK, 128), lambda i: (i, 0)),
            pl.BlockSpec((_EBLK, 128), lambda i: (i, 0)),
            pl.BlockSpec((1, 128), lambda i: (0, 0)),
        ],
        out_specs=pl.BlockSpec((_EBLK, 128), lambda i: (i, 0)),
        out_shape=jax.ShapeDtypeStruct((m, 128), jnp.float32),
    )(lp, rp, att128)


def _tcD_body(a0_ref, a1_ref, bias_ref, x1_ref, x2_ref,
              wv1_ref, bv1_ref, wva_ref, bva_ref, wvv_ref, bvv_ref,
              xc_ref, part_ref):
    a = a0_ref[...] + a1_ref[...]
    blk = a.shape[0]
    xc = jax.nn.relu(a[:, :10] / (a[:, 10:11] + 1e-16) + bias_ref[...])
    xc_ref[...] = jnp.concatenate([xc, jnp.zeros((blk, 6), jnp.float32)], axis=1)
    h = jnp.concatenate(
        [xc, x1_ref[...], jnp.broadcast_to(x2_ref[...], (blk, 4))], axis=1)
    V1 = jax.nn.relu(jnp.dot(h, wv1_ref[...], preferred_element_type=jnp.float32)
                     + bv1_ref[...])
    s = jnp.dot(V1, wva_ref[...], preferred_element_type=jnp.float32) + bva_ref[...]
    se = jnp.exp(s)
    sv = jnp.dot(V1, wvv_ref[...], preferred_element_type=jnp.float32) + bvv_ref[...]
    psum = jnp.sum(se * sv, axis=0, keepdims=True)
    tot = jnp.sum(se, axis=0, keepdims=True)
    part = jnp.concatenate([psum, tot, jnp.zeros((1, 5), jnp.float32)], axis=1)
    part_ref[...] = part.reshape(1, 1, 16)


def _tc_value(a0, a1, bias, x1, x2, p):
    n = x1.shape[0]
    grid = n // _BLK
    outs = [jax.ShapeDtypeStruct((n, 16), jnp.float32),
            jax.ShapeDtypeStruct((grid, 1, 16), jnp.float32)]
    return pl.pallas_call(
        _tcD_body,
        grid=(grid,),
        in_specs=[
            pl.BlockSpec((_BLK, 16), lambda i: (i, 0)),
            pl.BlockSpec((_BLK, 16), lambda i: (i, 0)),
            pl.BlockSpec((10,), lambda i: (0,)),
            pl.BlockSpec((_BLK, 15), lambda i: (i, 0)),
            pl.BlockSpec((1, 4), lambda i: (0, 0)),
            pl.BlockSpec((29, 20), lambda i: (0, 0)),
            pl.BlockSpec((20,), lambda i: (0,)),
            pl.BlockSpec((20, 1), lambda i: (0, 0)),
            pl.BlockSpec((1,), lambda i: (0,)),
            pl.BlockSpec((20, 10), lambda i: (0, 0)),
            pl.BlockSpec((10,), lambda i: (0,)),
        ],
        out_specs=[pl.BlockSpec((_BLK, 16), lambda i: (i, 0)),
                   pl.BlockSpec((1, 1, 16), lambda i: (i, 0, 0))],
        out_shape=outs,
    )(a0, a1, bias, x1, x2, p['Wv1'], p['bv1'], p['Wva'], p['bva'],
      p['Wvv'], p['bvv'])


def _tcE_body(parts_ref, gxc_ref, gx1_ref, aarm_ref, darm_ref,
              wvl_ref, bvl_ref, wa_ref, ba_ref, wd_ref, bd_ref,
              woa_ref, boa_ref, wov_ref, bov_ref,
              outv_ref, outpv_ref):
    parts = parts_ref[...][:, 0, :]
    sv = jnp.sum(parts[:, :10], axis=0, keepdims=True)
    tot = jnp.sum(parts[:, 10])
    V10 = jax.nn.relu(sv / tot)
    Vs = jnp.tanh(jnp.dot(V10, wvl_ref[...],
                          preferred_element_type=jnp.float32) + bvl_ref[...])
    outv_ref[...] = Vs

    gxc = gxc_ref[...]
    gx1 = gx1_ref[...]
    asrc_c = gxc[0:512].reshape(16, 32, 16)[..., :10]
    adst_c = gxc[512:1024].reshape(16, 32, 16)[..., :10]
    dtgt_c = gxc[1024:1280].reshape(16, 16, 16)[..., :10]
    asrc_1 = gx1[0:512].reshape(16, 32, 16)
    adst_1 = gx1[512:1024].reshape(16, 32, 16)
    dtgt_1 = gx1[1024:1280].reshape(16, 16, 16)
    aarm = aarm_ref[...]
    darm = darm_ref[...]
    extra2 = 0.6 * aarm - 0.7 * (adst_1[..., 3] + adst_1[..., 4])
    at = jnp.concatenate(
        [asrc_c, adst_c, asrc_1[..., 3:15], adst_1[..., 1:15],
         aarm[..., None], extra2[..., None]], axis=2)
    at = (jnp.dot(at.reshape(512, 48), wa_ref[...],
                  preferred_element_type=jnp.float32) + ba_ref[...]).reshape(16, 32, 20)
    dt = jnp.concatenate(
        [dtgt_c, dtgt_1[..., 3:15], darm[..., None]], axis=2)
    dt = (jnp.dot(dt.reshape(256, 23), wd_ref[...],
                  preferred_element_type=jnp.float32) + bd_ref[...]).reshape(16, 16, 20)
    ot = jax.nn.relu(jnp.concatenate([at, dt], axis=1))
    ot2 = ot.reshape(768, 20)
    oa = (jnp.dot(ot2, woa_ref[...],
                  preferred_element_type=jnp.float32) + boa_ref[...]).reshape(16, 48)
    ov = (jnp.dot(ot2, wov_ref[...],
                  preferred_element_type=jnp.float32) + bov_ref[...]).reshape(16, 48)
    attn = jax.nn.softmax(oa, axis=1)
    pv = jnp.sum(attn * ov, axis=1, keepdims=True).reshape(1, 16)
    outpv_ref[...] = jax.nn.log_softmax(pv, axis=1)


def _tc_heads(parts, gxc, gx1, aarm, darm, p):
    outs = [jax.ShapeDtypeStruct((1, 1), jnp.float32),
            jax.ShapeDtypeStruct((1, 16), jnp.float32)]
    return pl.pallas_call(
        _tcE_body,
        out_shape=outs,
    )(parts, gxc, gx1, aarm, darm,
      p['Wvl'], p['bvl'], p['Wa'], p['ba'], p['Wd'], p['bd'],
      p['Woa'], p['boa'], p['Wov'], p['bov'])


_CBLK = 25
_NS = 5


def _edge_gather(src, dst, xl, xr):
    e = src.shape[0]
    epw = e // _NW
    nblk = epw // _EPB
    nchunks = nblk // _CBLK
    cbe = _CBLK * _EPB
    mesh = plsc.VectorSubcoreMesh(core_axis_name="c", subcore_axis_name="s")

    scratch = [
        pltpu.VMEM((cbe,), jnp.int32),
        pltpu.VMEM((cbe,), jnp.int32),
    ]
    scratch += [pltpu.VMEM((_EPB, 16), jnp.float32) for _ in range(2 * _NS)]
    scratch += [pltpu.VMEM((_EPB * 16,), jnp.float32) for _ in range(2 * _NS)]
    scratch += [pltpu.SemaphoreType.DMA for _ in range(2 * _NS)]

    @functools.partial(
        pl.kernel,
        mesh=mesh,
        compiler_params=pltpu.CompilerParams(use_tc_tiling_on_sc=False),
        out_type=[jax.ShapeDtypeStruct((e * 16,), jnp.float32),
                  jax.ShapeDtypeStruct((e * 16,), jnp.float32)],
        scratch_types=scratch,
    )
    def gk(src_h, dst_h, xl_h, xr_h, outl_h, outr_h, *scr):
        srcs_big, dsts_big = scr[0], scr[1]
        rows_l = list(scr[2:2 + _NS])
        rows_r = list(scr[2 + _NS:2 + 2 * _NS])
        fl = list(scr[2 + 2 * _NS:2 + 3 * _NS])
        fr = list(scr[2 + 3 * _NS:2 + 4 * _NS])
        sem_g = list(scr[2 + 4 * _NS:2 + 5 * _NS])
        sem_w = list(scr[2 + 5 * _NS:2 + 6 * _NS])
        cid = lax.axis_index("c")
        sid = lax.axis_index("s")
        wid = sid * 2 + cid
        base0 = wid * epw

        def issue_gather(b, k):
            o = pl.multiple_of(k * _EPB, 16)
            g1 = pltpu.async_copy(xl_h.at[dsts_big.at[pl.ds(o, _EPB)]],
                                  rows_l[b], sem_g[b])
            g2 = pltpu.async_copy(xr_h.at[srcs_big.at[pl.ds(o, _EPB)]],
                                  rows_r[b], sem_g[b])
            return g1, g2

        def wait_gather(b):
            pltpu.make_async_copy(xl_h.at[dsts_big.at[pl.ds(0, _EPB)]],
                                  rows_l[b], sem_g[b]).wait()
            pltpu.make_async_copy(xr_h.at[srcs_big.at[pl.ds(0, _EPB)]],
                                  rows_r[b], sem_g[b]).wait()

        def wait_writeout(b):
            pltpu.make_async_copy(fl[b], outl_h.at[pl.ds(0, _EPB * 16)],
                                  sem_w[b]).wait()
            pltpu.make_async_copy(fr[b], outr_h.at[pl.ds(0, _EPB * 16)],
                                  sem_w[b]).wait()

        def chunk(c, carry):
            cb = pl.multiple_of(base0 + c * cbe, 16)
            pltpu.sync_copy(src_h.at[pl.ds(cb, cbe)], srcs_big)
            pltpu.sync_copy(dst_h.at[pl.ds(cb, cbe)], dsts_big)
            for b in range(_NS):
                issue_gather(b, b)

            def group(g, carry2):
                for b in range(_NS):
                    k = g * _NS + b
                    gi = c * _CBLK + k
                    wait_gather(b)
                    @pl.when(gi >= _NS)
                    def _():
                        wait_writeout(b)
                    for j in range(_EPB):
                        fl[b][pl.ds(j * 16, 16)] = rows_l[b][j]
                        fr[b][pl.ds(j * 16, 16)] = rows_r[b][j]
                    fbase = pl.multiple_of((base0 + gi * _EPB) * 16, 128)
                    pltpu.async_copy(fl[b], outl_h.at[pl.ds(fbase, _EPB * 16)],
                                     sem_w[b])
                    pltpu.async_copy(fr[b], outr_h.at[pl.ds(fbase, _EPB * 16)],
                                     sem_w[b])
                    @pl.when(k + _NS < _CBLK)
                    def _():
                        issue_gather(b, k + _NS)
                return carry2
            lax.fori_loop(0, _CBLK // _NS, group, 0)
            return carry
        lax.fori_loop(0, nchunks, chunk, 0)
        for b in range(_NS):
            wait_writeout(b)

    return gk(src, dst, xl, xr)


def _edge_scatter(dst, vflat, n):
    e = dst.shape[0]
    epw = e // _NW
    nblk = epw // _EPB
    nchunks = nblk // _CBLK
    rows_sub = _NPAD // 16
    chunk_rows = 400
    nchunk = rows_sub // chunk_rows
    mesh = plsc.VectorSubcoreMesh(core_axis_name="c", subcore_axis_name="s")

    scratch = [pltpu.VMEM((_CBLK * _EPB,), jnp.int32)]
    scratch += [pltpu.VMEM((_EPB * 16,), jnp.float32) for _ in range(_NS)]
    scratch += [pltpu.VMEM((_EPB, 16), jnp.float32) for _ in range(_NS)]
    scratch += [pltpu.VMEM((_EPB,), jnp.int32) for _ in range(_NS)]
    scratch += [
        pltpu.VMEM((chunk_rows, 16), jnp.float32),
        pltpu.VMEM_SHARED((_NPAD, 16), jnp.float32),
    ]
    scratch += [pltpu.SemaphoreType.DMA for _ in range(2 * _NS)]

    @functools.partial(
        pl.kernel,
        mesh=mesh,
        compiler_params=pltpu.CompilerParams(use_tc_tiling_on_sc=False),
        out_type=jax.ShapeDtypeStruct((2 * _NPAD, 16), jnp.float32),
        scratch_types=scratch,
    )
    def sk(dst_h, v_h, out_h, *scr):
        dsts_big = scr[0]
        vbuf = list(scr[1:1 + _NS])
        sbuf = list(scr[1 + _NS:1 + 2 * _NS])
        dstc = list(scr[1 + 2 * _NS:1 + 3 * _NS])
        stage = scr[1 + 3 * _NS]
        acc_sh = scr[2 + 3 * _NS]
        sem_v = list(scr[3 + 3 * _NS:3 + 4 * _NS])
        sem_s = list(scr[3 + 4 * _NS:3 + 5 * _NS])
        cid = lax.axis_index("c")
        sid = lax.axis_index("s")
        wid = sid * 2 + cid

        zv = jnp.zeros((16,), jnp.float32)

        def zrow(r, carry):
            stage[r] = zv
            return carry
        lax.fori_loop(0, chunk_rows, zrow, 0)

        def zchunk(c, carry):
            r0 = pl.multiple_of(sid * rows_sub + c * chunk_rows, 8)
            pltpu.sync_copy(stage, acc_sh.at[pl.ds(r0, chunk_rows)])
            return carry
        lax.fori_loop(0, nchunk, zchunk, 0)
        plsc.subcore_barrier()

        base0 = wid * epw

        def issue_vload(b, gi):
            fbase = pl.multiple_of((base0 + gi * _EPB) * 16, 128)
            pltpu.async_copy(v_h.at[pl.ds(fbase, _EPB * 16)], vbuf[b], sem_v[b])

        def wait_vload(b):
            pltpu.make_async_copy(v_h.at[pl.ds(0, _EPB * 16)], vbuf[b],
                                  sem_v[b]).wait()

        def wait_scatter(b):
            pltpu.make_async_copy(sbuf[b], acc_sh.at[dstc[b]], sem_s[b]).wait()

        def chunk(c, carry):
            cb = pl.multiple_of(base0 + c * _CBLK * _EPB, 16)
            pltpu.sync_copy(dst_h.at[pl.ds(cb, _CBLK * _EPB)], dsts_big)
            for b in range(_NS):
                issue_vload(b, c * _CBLK + b)

            def group(g, carry2):
                for b in range(_NS):
                    k = g * _NS + b
                    gi = c * _CBLK + k
                    wait_vload(b)
                    @pl.when(gi >= _NS)
                    def _():
                        wait_scatter(b)
                    for j in range(_EPB):
                        sbuf[b][j] = vbuf[b][pl.ds(j * 16, 16)]
                    ko = pl.multiple_of(k * _EPB, 16)
                    for j5 in range(_EPB // 16):
                        dstc[b][pl.ds(j5 * 16, 16)] = (
                            dsts_big[pl.ds(ko + j5 * 16, 16)])
                    pltpu.async_copy(sbuf[b], acc_sh.at[dstc[b]], sem_s[b],
                                     add=True)
                    @pl.when(k + _NS < _CBLK)
                    def _():
                        issue_vload(b, gi + _NS)
                return carry2
            lax.fori_loop(0, _CBLK // _NS, group, 0)
            return carry
        lax.fori_loop(0, nchunks, chunk, 0)
        for b in range(_NS):
            wait_scatter(b)
        plsc.subcore_barrier()

        def dchunk(c, carry):
            r0 = pl.multiple_of(sid * rows_sub + c * chunk_rows, 8)
            pltpu.sync_copy(acc_sh.at[pl.ds(r0, chunk_rows)], stage)
            pltpu.sync_copy(stage, out_h.at[pl.ds(cid * _NPAD + r0, chunk_rows)])
            return carry
        lax.fori_loop(0, nchunk, dchunk, 0)

    return sk(dst, vflat)


def _edge_fused(src, dst, xl, xr, attp):
    e = src.shape[0]
    epw = e // _NW
    nblk = epw // _EPB
    nchunks = nblk // _CBLK
    cbe = _CBLK * _EPB
    rows_sub = _NPAD // 16
    chunk_rows = 400
    nchunk = rows_sub // chunk_rows
    mesh = plsc.VectorSubcoreMesh(core_axis_name="c", subcore_axis_name="s")

    scratch = [
        pltpu.VMEM((cbe,), jnp.int32),
        pltpu.VMEM((cbe,), jnp.int32),
        pltpu.VMEM((16,), jnp.float32),
        pltpu.VMEM((chunk_rows, 16), jnp.float32),
        pltpu.VMEM_SHARED((_NPAD, 16), jnp.float32),
    ]
    scratch += [pltpu.VMEM((_EPB, 16), jnp.float32) for _ in range(3 * _NS)]
    scratch += [pltpu.VMEM((_EPB,), jnp.int32) for _ in range(_NS)]
    scratch += [pltpu.SemaphoreType.DMA for _ in range(2 * _NS)]

    @functools.partial(
        pl.kernel,
        mesh=mesh,
        compiler_params=pltpu.CompilerParams(
            use_tc_tiling_on_sc=False, needs_layout_passes=False),
        out_type=jax.ShapeDtypeStruct((2 * _NPAD, 16), jnp.float32),
        scratch_types=scratch,
    )
    def ek(src_h, dst_h, xl_h, xr_h, att_h, out_h, *scr):
        srcs_big, dsts_big, att_v, stage, acc_sh = scr[:5]
        rows_l = list(scr[5:5 + _NS])
        rows_r = list(scr[5 + _NS:5 + 2 * _NS])
        sbuf = list(scr[5 + 2 * _NS:5 + 3 * _NS])
        dstc = list(scr[5 + 3 * _NS:5 + 4 * _NS])
        sem_g = list(scr[5 + 4 * _NS:5 + 5 * _NS])
        sem_s = list(scr[5 + 5 * _NS:5 + 6 * _NS])
        cid = lax.axis_index("c")
        sid = lax.axis_index("s")
        wid = sid * 2 + cid
        base0 = wid * epw

        zv = jnp.zeros((16,), jnp.float32)

        def zrow(r, carry):
            stage[r] = zv
            return carry
        lax.fori_loop(0, chunk_rows, zrow, 0)

        def zchunk(c, carry):
            r0 = pl.multiple_of(sid * rows_sub + c * chunk_rows, 8)
            pltpu.sync_copy(stage, acc_sh.at[pl.ds(r0, chunk_rows)])
            return carry
        lax.fori_loop(0, nchunk, zchunk, 0)
        plsc.subcore_barrier()

        pltpu.sync_copy(att_h, att_v)
        att = att_v[...]
        att06 = att * 0.6
        att04 = att * 0.4

        def issue_gather(b, k):
            o = pl.multiple_of(k * _EPB, 16)
            pltpu.async_copy(xl_h.at[dsts_big.at[pl.ds(o, _EPB)]],
                             rows_l[b], sem_g[b])
            pltpu.async_copy(xr_h.at[srcs_big.at[pl.ds(o, _EPB)]],
                             rows_r[b], sem_g[b])

        def wait_gather(b):
            pltpu.make_async_copy(xl_h.at[dsts_big.at[pl.ds(0, _EPB)]],
                                  rows_l[b], sem_g[b]).wait()
            pltpu.make_async_copy(xr_h.at[srcs_big.at[pl.ds(0, _EPB)]],
                                  rows_r[b], sem_g[b]).wait()

        def wait_scatter(b):
            pltpu.make_async_copy(sbuf[b], acc_sh.at[dstc[b]], sem_s[b]).wait()

        def chunk(c, carry):
            cb = pl.multiple_of(base0 + c * cbe, 16)
            pltpu.sync_copy(src_h.at[pl.ds(cb, cbe)], srcs_big)
            pltpu.sync_copy(dst_h.at[pl.ds(cb, cbe)], dsts_big)
            for b in range(_NS):
                issue_gather(b, b)

            def group(g, carry2):
                for b in range(_NS):
                    k = g * _NS + b
                    gi = c * _CBLK + k
                    wait_gather(b)
                    @pl.when(gi >= _NS)
                    def _():
                        wait_scatter(b)
                    for j in range(_EPB):
                        rr = rows_r[b][j]
                        s = rows_l[b][j] + rr
                        t = s * att06 + jnp.abs(s) * att04
                        lj = jnp.sum(t)
                        exv = jnp.exp(jnp.broadcast_to(lj, (16,)))
                        sbuf[b][j] = rr * exv
                    ko = pl.multiple_of(k * _EPB, 16)
                    for j5 in range(_EPB // 16):
                        dstc[b][pl.ds(j5 * 16, 16)] = (
                            dsts_big[pl.ds(ko + j5 * 16, 16)])
                    pltpu.async_copy(sbuf[b], acc_sh.at[dstc[b]], sem_s[b],
                                     add=True)
                    @pl.when(k + _NS < _CBLK)
                    def _():
                        issue_gather(b, k + _NS)
                return carry2
            lax.fori_loop(0, _CBLK // _NS, group, 0)
            return carry
        lax.fori_loop(0, nchunks, chunk, 0)
        for b in range(_NS):
            wait_scatter(b)
        plsc.subcore_barrier()

        def dchunk(c, carry):
            r0 = pl.multiple_of(sid * rows_sub + c * chunk_rows, 8)
            pltpu.sync_copy(acc_sh.at[pl.ds(r0, chunk_rows)], stage)
            pltpu.sync_copy(stage, out_h.at[pl.ds(cid * _NPAD + r0, chunk_rows)])
            return carry
        lax.fori_loop(0, nchunk, dchunk, 0)

    return ek(src, dst, xl, xr, attp)


def _move_gather(idx, xcp, x1p):
    b = idx.shape[0]
    per_w = b // _NW
    mesh = plsc.VectorSubcoreMesh(core_axis_name="c", subcore_axis_name="s")

    @functools.partial(
        pl.kernel,
        mesh=mesh,
        compiler_params=pltpu.CompilerParams(use_tc_tiling_on_sc=False),
        out_type=[jax.ShapeDtypeStruct((b, 16), jnp.float32),
                  jax.ShapeDtypeStruct((b, 16), jnp.float32)],
        scratch_types=[
            pltpu.VMEM((per_w,), jnp.int32),
            pltpu.VMEM((per_w, 16), jnp.float32),
            pltpu.VMEM((per_w, 16), jnp.float32),
            pltpu.SemaphoreType.DMA,
            pltpu.SemaphoreType.DMA,
        ],
    )
    def mg(idx_h, xc_h, x1_h, oxc_h, ox1_h, idxv, r1, r2, sem1, sem2):
        cid = lax.axis_index("c")
        sid = lax.axis_index("s")
        wid = sid * 2 + cid
        base = pl.multiple_of(wid * per_w, 8)
        pltpu.sync_copy(idx_h.at[pl.ds(base, per_w)], idxv)
        a = pltpu.async_copy(xc_h.at[idxv], r1, sem1)
        bb = pltpu.async_copy(x1_h.at[idxv], r2, sem2)
        a.wait()
        bb.wait()
        pltpu.sync_copy(r1, oxc_h.at[pl.ds(base, per_w)])
        pltpu.sync_copy(r2, ox1_h.at[pl.ds(base, per_w)])

    return mg(idx, xcp, x1p)


def _edge_pass(src, dst, xl, xr, attp, n):
    return _edge_fused(src, dst, xl, xr, attp)


def kernel(x1, x2, attack_armies, deploy_armies, params, edges, attack_src,
           attack_dst, deploy_target):
    p = params
    n = x1.shape[0]
    src = edges[0]
    dst = edges[1]

    Wlr1 = jnp.concatenate([p['Wl1'], p['Wr1']], axis=1)
    Wlr2 = jnp.concatenate([p['Wl2'], p['Wr2']], axis=1)
    Wlr3 = jnp.concatenate([p['Wl3'], p['Wr3']], axis=1)
    att1p = jnp.pad(p['att1'], (0, 6))
    att2p = jnp.pad(p['att2'], (0, 6))
    att3p = jnp.pad(p['att3'], (0, 6))
    x1p = jnp.pad(x1, ((0, 0), (0, 1)))

    xp, xl1, xr1 = _tc_first(x1, p['W0'], p['b0'], Wlr1)
    acc1 = _edge_pass(src, dst, xl1, xr1, att1p, n)
    xap, xl2, xr2 = _tc_mid(acc1[:n], acc1[_NPAD:_NPAD + n], p['bias1'],
                            x1, [xp], Wlr2)
    acc2 = _edge_pass(src, dst, xl2, xr2, att2p, n)
    xbp, xl3, xr3 = _tc_mid(acc2[:n], acc2[_NPAD:_NPAD + n], p['bias2'],
                            x1, [xap, xp], Wlr3)
    acc3 = _edge_pass(src, dst, xl3, xr3, att3p, n)

    xcp, parts = _tc_value(acc3[:n], acc3[_NPAD:_NPAD + n], p['bias3'],
                           x1, x2, p)

    midx = jnp.concatenate([attack_src.reshape(-1), attack_dst.reshape(-1),
                            deploy_target.reshape(-1)])
    gxc, gx1 = _move_gather(midx, xcp, x1p)

    outv, outpv = _tc_heads(parts, gxc, gx1, attack_armies, deploy_armies, p)
    return (outv.reshape(()), outpv.reshape(16))
